# Initial kernel scaffold; baseline (speedup 1.0000x reference)
import jax
import jax.numpy as jnp
from jax.experimental import pallas as pl


def kernel(boxes, gt_boxes, gt_classes):
    raise NotImplementedError("write your pallas kernel here")


# speedup vs baseline: 5.9428x; 5.9428x over previous
import jax
import jax.numpy as jnp
from jax.experimental import pallas as pl

_NUM_SAMPLED = 512
_FG_FRACTION = 0.25
_FG_IOU = 0.5
_TILE_N = 2048
_GT_PAD = 128


def _match_kernel(num_gt, boxes_ref, gt_ref, rand_ref, vals_ref, idx_ref,
                  score_ref):
    bt = boxes_ref[0]
    g = gt_ref[0]
    y1a = bt[0:1, :]
    x1a = bt[1:2, :]
    y2a = bt[2:3, :]
    x2a = bt[3:4, :]
    y1b = g[:, 0:1]
    x1b = g[:, 1:2]
    y2b = g[:, 2:3]
    x2b = g[:, 3:4]
    inter_h = jnp.maximum(jnp.minimum(y2a, y2b) - jnp.maximum(y1a, y1b), 0.0)
    inter_w = jnp.maximum(jnp.minimum(x2a, x2b) - jnp.maximum(x1a, x1b), 0.0)
    inter = inter_h * inter_w
    area1 = (y2a - y1a) * (x2a - x1a)
    area2 = (y2b - y1b) * (x2b - x1b)
    union = area1 + area2 - inter
    safe_union = jnp.where(union > 0.0, union, 1.0)
    sim = jnp.where(union > 0.0, inter / safe_union, 0.0)
    row = jax.lax.broadcasted_iota(jnp.int32, sim.shape, 0)
    sim = jnp.where(row < num_gt, sim, -1.0)
    vals = jnp.max(sim, axis=0, keepdims=True)
    hit = sim == vals
    idx = jnp.min(jnp.where(hit, row, _GT_PAD), axis=0, keepdims=True)
    vals_ref[0, 0] = vals
    idx_ref[0, 0] = idx
    rand = rand_ref[0, 0]
    score_ref[0, 0] = jnp.where(vals >= _FG_IOU, rand, -1.0)


def kernel(boxes, gt_boxes, gt_classes):
    gt_boxes = gt_boxes.astype(boxes.dtype)
    boxes = jnp.concatenate([boxes, gt_boxes], axis=1)
    B, N, _ = boxes.shape
    M = gt_boxes.shape[1]
    n_pad = ((N + _TILE_N - 1) // _TILE_N) * _TILE_N
    boxes_t = jnp.pad(jnp.swapaxes(boxes, 1, 2), ((0, 0), (0, 0), (0, n_pad - N)))
    gt_p = jnp.pad(gt_boxes, ((0, 0), (0, _GT_PAD - M), (0, 0)))
    rand = jax.random.uniform(jax.random.key(42), (B, N))
    rand_p = jnp.pad(rand, ((0, 0), (0, n_pad - N)))

    nt = n_pad // _TILE_N
    rand4 = rand_p.reshape(B, nt, 1, _TILE_N)
    grid = (B, nt)
    out_shape = [
        jax.ShapeDtypeStruct((B, nt, 1, _TILE_N), jnp.float32),
        jax.ShapeDtypeStruct((B, nt, 1, _TILE_N), jnp.int32),
        jax.ShapeDtypeStruct((B, nt, 1, _TILE_N), jnp.float32),
    ]
    vec_spec = pl.BlockSpec((1, 1, 1, _TILE_N), lambda b, t: (b, t, 0, 0))
    vals, midx, pos_score = pl.pallas_call(
        lambda *refs: _match_kernel(M, *refs),
        grid=grid,
        in_specs=[
            pl.BlockSpec((1, 4, _TILE_N), lambda b, t: (b, 0, t)),
            pl.BlockSpec((1, _GT_PAD, 4), lambda b, t: (b, 0, 0)),
            vec_spec,
        ],
        out_specs=[vec_spec, vec_spec, vec_spec],
        out_shape=out_shape,
    )(boxes_t, gt_p, rand4)

    vals = vals.reshape(B, n_pad)[:, :N]
    midx = midx.reshape(B, n_pad)[:, :N]
    pos_score = pos_score.reshape(B, n_pad)[:, :N]

    positive = vals >= _FG_IOU
    bg = jnp.logical_not(positive)

    max_pos = int(_NUM_SAMPLED * _FG_FRACTION)
    top_vals, _ = jax.lax.top_k(pos_score, max_pos)
    kth = top_vals[:, -1:]
    sampled_pos = positive & (pos_score >= jnp.maximum(kth, 0.0))
    combined = jnp.where(sampled_pos, rand + 2.0, jnp.where(bg, rand, -1.0))
    _, indices = jax.lax.top_k(combined, _NUM_SAMPLED)

    rois = jnp.take_along_axis(boxes, indices[..., None], axis=1)
    s_midx = jnp.take_along_axis(midx, indices, axis=1)
    s_bg = jnp.take_along_axis(bg, indices, axis=1)
    s_gt_boxes = jnp.take_along_axis(gt_boxes, s_midx[..., None], axis=1)
    s_gt_boxes = jnp.where(s_bg[..., None], 0.0, s_gt_boxes)
    s_gt_classes = jnp.take_along_axis(gt_classes, s_midx, axis=1)
    s_gt_classes = jnp.where(s_bg, 0, s_gt_classes)
    s_gt_indices = jnp.where(s_bg, -1, s_midx)
    return rois, s_gt_boxes, s_gt_classes, s_gt_indices


# speedup vs baseline: 8.1661x; 1.3741x over previous
import jax
import jax.numpy as jnp
from jax.experimental import pallas as pl

_NUM_SAMPLED = 512
_FG_FRACTION = 0.25
_FG_IOU = 0.5
_TILE_N = 2048
_GT_PAD = 128


def _match_kernel(num_gt, boxes_ref, gt_ref, rand_ref, vals_ref, idx_ref,
                  score_ref):
    bt = boxes_ref[0]
    g = gt_ref[0]
    y1a = bt[0:1, :]
    x1a = bt[1:2, :]
    y2a = bt[2:3, :]
    x2a = bt[3:4, :]
    y1b = g[:, 0:1]
    x1b = g[:, 1:2]
    y2b = g[:, 2:3]
    x2b = g[:, 3:4]
    inter_h = jnp.maximum(jnp.minimum(y2a, y2b) - jnp.maximum(y1a, y1b), 0.0)
    inter_w = jnp.maximum(jnp.minimum(x2a, x2b) - jnp.maximum(x1a, x1b), 0.0)
    inter = inter_h * inter_w
    area1 = (y2a - y1a) * (x2a - x1a)
    area2 = (y2b - y1b) * (x2b - x1b)
    union = area1 + area2 - inter
    safe_union = jnp.where(union > 0.0, union, 1.0)
    sim = jnp.where(union > 0.0, inter / safe_union, 0.0)
    row = jax.lax.broadcasted_iota(jnp.int32, sim.shape, 0)
    sim = jnp.where(row < num_gt, sim, -1.0)
    vals = jnp.max(sim, axis=0, keepdims=True)
    hit = sim == vals
    idx = jnp.min(jnp.where(hit, row, _GT_PAD), axis=0, keepdims=True)
    vals_ref[0, 0] = vals
    idx_ref[0, 0] = idx
    rand = rand_ref[0, 0]
    score_ref[0, 0] = jnp.where(vals >= _FG_IOU, rand, -1.0)


def kernel(boxes, gt_boxes, gt_classes):
    gt_boxes = gt_boxes.astype(boxes.dtype)
    boxes = jnp.concatenate([boxes, gt_boxes], axis=1)
    B, N, _ = boxes.shape
    M = gt_boxes.shape[1]
    n_pad = ((N + _TILE_N - 1) // _TILE_N) * _TILE_N
    boxes_t = jnp.pad(jnp.swapaxes(boxes, 1, 2), ((0, 0), (0, 0), (0, n_pad - N)))
    gt_p = jnp.pad(gt_boxes, ((0, 0), (0, _GT_PAD - M), (0, 0)))
    rand = jax.random.uniform(jax.random.key(42), (B, N))
    rand_p = jnp.pad(rand, ((0, 0), (0, n_pad - N)))

    nt = n_pad // _TILE_N
    rand4 = rand_p.reshape(B, nt, 1, _TILE_N)
    grid = (B, nt)
    out_shape = [
        jax.ShapeDtypeStruct((B, nt, 1, _TILE_N), jnp.float32),
        jax.ShapeDtypeStruct((B, nt, 1, _TILE_N), jnp.int32),
        jax.ShapeDtypeStruct((B, nt, 1, _TILE_N), jnp.float32),
    ]
    vec_spec = pl.BlockSpec((1, 1, 1, _TILE_N), lambda b, t: (b, t, 0, 0))
    vals, midx, pos_score = pl.pallas_call(
        lambda *refs: _match_kernel(M, *refs),
        grid=grid,
        in_specs=[
            pl.BlockSpec((1, 4, _TILE_N), lambda b, t: (b, 0, t)),
            pl.BlockSpec((1, _GT_PAD, 4), lambda b, t: (b, 0, 0)),
            vec_spec,
        ],
        out_specs=[vec_spec, vec_spec, vec_spec],
        out_shape=out_shape,
    )(boxes_t, gt_p, rand4)

    vals = vals.reshape(B, n_pad)[:, :N]
    midx = midx.reshape(B, n_pad)[:, :N]
    pos_score = pos_score.reshape(B, n_pad)[:, :N]

    positive = vals >= _FG_IOU
    bg = jnp.logical_not(positive)

    n_chunks = 10
    chunk = N // n_chunks
    max_pos = int(_NUM_SAMPLED * _FG_FRACTION)
    ps_c = pos_score.reshape(B, n_chunks, chunk)
    pv, _ = jax.lax.top_k(ps_c, max_pos)
    top_vals, _ = jax.lax.top_k(pv.reshape(B, n_chunks * max_pos), max_pos)
    kth = top_vals[:, -1:]
    sampled_pos = positive & (pos_score >= jnp.maximum(kth, 0.0))
    combined = jnp.where(sampled_pos, rand + 2.0, jnp.where(bg, rand, -1.0))
    cb_c = combined.reshape(B, n_chunks, chunk)
    cv, ci = jax.lax.top_k(cb_c, _NUM_SAMPLED)
    base = (jnp.arange(n_chunks, dtype=jnp.int32) * chunk)[None, :, None]
    gi = (ci + base).reshape(B, n_chunks * _NUM_SAMPLED)
    _, pos_sel = jax.lax.top_k(cv.reshape(B, n_chunks * _NUM_SAMPLED),
                               _NUM_SAMPLED)
    indices = jnp.take_along_axis(gi, pos_sel, axis=1)

    rois = jnp.take_along_axis(boxes, indices[..., None], axis=1)
    s_midx = jnp.take_along_axis(midx, indices, axis=1)
    s_bg = jnp.take_along_axis(bg, indices, axis=1)
    s_gt_boxes = jnp.take_along_axis(gt_boxes, s_midx[..., None], axis=1)
    s_gt_boxes = jnp.where(s_bg[..., None], 0.0, s_gt_boxes)
    s_gt_classes = jnp.take_along_axis(gt_classes, s_midx, axis=1)
    s_gt_classes = jnp.where(s_bg, 0, s_gt_classes)
    s_gt_indices = jnp.where(s_bg, -1, s_midx)
    return rois, s_gt_boxes, s_gt_classes, s_gt_indices


# speedup vs baseline: 8.3967x; 1.0282x over previous
import jax
import jax.numpy as jnp
from jax.experimental import pallas as pl

_NUM_SAMPLED = 512
_FG_FRACTION = 0.25
_FG_IOU = 0.5
_TILE_N = 2048
_GT_PAD = 128


def _match_kernel(num_gt, boxes_ref, gt_ref, rand_ref, vals_ref, idx_ref,
                  score_ref):
    bt = boxes_ref[0]
    g = gt_ref[0]
    y1a = bt[0:1, :]
    x1a = bt[1:2, :]
    y2a = bt[2:3, :]
    x2a = bt[3:4, :]
    y1b = g[:, 0:1]
    x1b = g[:, 1:2]
    y2b = g[:, 2:3]
    x2b = g[:, 3:4]
    inter_h = jnp.maximum(jnp.minimum(y2a, y2b) - jnp.maximum(y1a, y1b), 0.0)
    inter_w = jnp.maximum(jnp.minimum(x2a, x2b) - jnp.maximum(x1a, x1b), 0.0)
    inter = inter_h * inter_w
    area1 = (y2a - y1a) * (x2a - x1a)
    area2 = (y2b - y1b) * (x2b - x1b)
    union = area1 + area2 - inter
    safe_union = jnp.where(union > 0.0, union, 1.0)
    sim = jnp.where(union > 0.0, inter / safe_union, 0.0)
    row = jax.lax.broadcasted_iota(jnp.int32, sim.shape, 0)
    sim = jnp.where(row < num_gt, sim, -1.0)
    vals = jnp.max(sim, axis=0, keepdims=True)
    hit = sim == vals
    idx = jnp.min(jnp.where(hit, row, _GT_PAD), axis=0, keepdims=True)
    vals_ref[0, 0] = vals
    idx_ref[0, 0] = idx
    rand = rand_ref[0, 0]
    score_ref[0, 0] = jnp.where(vals >= _FG_IOU, rand, -1.0)


def kernel(boxes, gt_boxes, gt_classes):
    gt_boxes = gt_boxes.astype(boxes.dtype)
    boxes = jnp.concatenate([boxes, gt_boxes], axis=1)
    B, N, _ = boxes.shape
    M = gt_boxes.shape[1]
    n_pad = ((N + _TILE_N - 1) // _TILE_N) * _TILE_N
    boxes_t = jnp.pad(jnp.swapaxes(boxes, 1, 2), ((0, 0), (0, 0), (0, n_pad - N)))
    gt_p = jnp.pad(gt_boxes, ((0, 0), (0, _GT_PAD - M), (0, 0)))
    rand = jax.random.uniform(jax.random.key(42), (B, N))
    rand_p = jnp.pad(rand, ((0, 0), (0, n_pad - N)))

    nt = n_pad // _TILE_N
    rand4 = rand_p.reshape(B, nt, 1, _TILE_N)
    grid = (B, nt)
    out_shape = [
        jax.ShapeDtypeStruct((B, nt, 1, _TILE_N), jnp.float32),
        jax.ShapeDtypeStruct((B, nt, 1, _TILE_N), jnp.int32),
        jax.ShapeDtypeStruct((B, nt, 1, _TILE_N), jnp.float32),
    ]
    vec_spec = pl.BlockSpec((1, 1, 1, _TILE_N), lambda b, t: (b, t, 0, 0))
    vals, midx, pos_score = pl.pallas_call(
        lambda *refs: _match_kernel(M, *refs),
        grid=grid,
        in_specs=[
            pl.BlockSpec((1, 4, _TILE_N), lambda b, t: (b, 0, t)),
            pl.BlockSpec((1, _GT_PAD, 4), lambda b, t: (b, 0, 0)),
            vec_spec,
        ],
        out_specs=[vec_spec, vec_spec, vec_spec],
        out_shape=out_shape,
    )(boxes_t, gt_p, rand4)

    vals = vals.reshape(B, n_pad)[:, :N]
    midx = midx.reshape(B, n_pad)[:, :N]
    pos_score = pos_score.reshape(B, n_pad)[:, :N]

    positive = vals >= _FG_IOU
    bg = jnp.logical_not(positive)

    nc_pos = 25
    nc_cmb = 5
    max_pos = int(_NUM_SAMPLED * _FG_FRACTION)
    ps_c = pos_score.reshape(B, nc_pos, N // nc_pos)
    pv, _ = jax.lax.top_k(ps_c, max_pos)
    top_vals, _ = jax.lax.top_k(pv.reshape(B, nc_pos * max_pos), max_pos)
    kth = top_vals[:, -1:]
    sampled_pos = positive & (pos_score >= jnp.maximum(kth, 0.0))
    combined = jnp.where(sampled_pos, rand + 2.0, jnp.where(bg, rand, -1.0))
    cb_c = combined.reshape(B, nc_cmb, N // nc_cmb)
    cv, ci = jax.lax.top_k(cb_c, _NUM_SAMPLED)
    base = (jnp.arange(nc_cmb, dtype=jnp.int32) * (N // nc_cmb))[None, :, None]
    gi = (ci + base).reshape(B, nc_cmb * _NUM_SAMPLED)
    _, pos_sel = jax.lax.top_k(cv.reshape(B, nc_cmb * _NUM_SAMPLED),
                               _NUM_SAMPLED)
    indices = jnp.take_along_axis(gi, pos_sel, axis=1)

    rois = jnp.take_along_axis(boxes, indices[..., None], axis=1)
    s_midx = jnp.take_along_axis(midx, indices, axis=1)
    s_bg = jnp.take_along_axis(bg, indices, axis=1)
    s_gt_boxes = jnp.take_along_axis(gt_boxes, s_midx[..., None], axis=1)
    s_gt_boxes = jnp.where(s_bg[..., None], 0.0, s_gt_boxes)
    s_gt_classes = jnp.take_along_axis(gt_classes, s_midx, axis=1)
    s_gt_classes = jnp.where(s_bg, 0, s_gt_classes)
    s_gt_indices = jnp.where(s_bg, -1, s_midx)
    return rois, s_gt_boxes, s_gt_classes, s_gt_indices
